# baseline (speedup 1.0000x reference)
import jax
import jax.numpy as jnp
from jax import lax
from jax.experimental import pallas as pl
from jax.experimental.pallas import tpu as pltpu
from jax.experimental.pallas import tpu_sc as plsc

N_HEADS = 16
HEAD_DIM = 128
N_TRAIN = 2048
S_NEW = 8192
S_OUT = N_TRAIN + S_NEW
HALF = S_OUT // 2


def _tc_assemble_k(tk_ref, nk_ref, ok_ref):
    ok_ref[0, 0, :N_TRAIN, :] = tk_ref[0, 0]
    ok_ref[0, 0, N_TRAIN:, :] = nk_ref[0, 0]


def _sc_assemble_v(tv_hbm, nv_hbm, out_hbm, sem):
    c = lax.axis_index("c")
    s = lax.axis_index("s")
    w = s * 2 + c
    h = w // 2
    even = (w % 2) == 0

    @pl.when(even)
    def _():
        c0 = pltpu.async_copy(
            tv_hbm.at[pl.ds(h * N_TRAIN, N_TRAIN), :],
            out_hbm.at[pl.ds(h * S_OUT, N_TRAIN), :], sem)
        c1 = pltpu.async_copy(
            nv_hbm.at[pl.ds(h * S_NEW, HALF - N_TRAIN), :],
            out_hbm.at[pl.ds(h * S_OUT + N_TRAIN, HALF - N_TRAIN), :], sem)
        c0.wait()
        c1.wait()

    @pl.when(jnp.logical_not(even))
    def _():
        c2 = pltpu.async_copy(
            nv_hbm.at[pl.ds(h * S_NEW + (HALF - N_TRAIN), HALF), :],
            out_hbm.at[pl.ds(h * S_OUT + HALF, HALF), :], sem)
        c2.wait()


def kernel(new_keys, new_values, trainable_keys, trainable_values,
           mem_keys, mem_values, new_seq_ids):
    del mem_keys, mem_values, new_seq_ids

    train_spec = pl.BlockSpec((1, 1, N_TRAIN, HEAD_DIM), lambda h: (0, h, 0, 0))
    new_spec = pl.BlockSpec((1, 1, S_NEW, HEAD_DIM), lambda h: (0, h, 0, 0))
    out_spec = pl.BlockSpec((1, 1, S_OUT, HEAD_DIM), lambda h: (0, h, 0, 0))
    out_shape = jax.ShapeDtypeStruct((1, N_HEADS, S_OUT, HEAD_DIM), jnp.float32)
    out_k = pl.pallas_call(
        _tc_assemble_k,
        grid=(N_HEADS,),
        in_specs=[train_spec, new_spec],
        out_specs=out_spec,
        out_shape=out_shape,
    )(trainable_keys, new_keys)

    tv2 = trainable_values.reshape(N_HEADS * N_TRAIN, HEAD_DIM)
    nv2 = new_values.reshape(N_HEADS * S_NEW, HEAD_DIM)
    out_v2 = pl.kernel(
        _sc_assemble_v,
        out_type=jax.ShapeDtypeStruct((N_HEADS * S_OUT, HEAD_DIM), jnp.float32),
        mesh=plsc.VectorSubcoreMesh(core_axis_name="c", subcore_axis_name="s"),
        scratch_types=[pltpu.SemaphoreType.DMA],
    )(tv2, nv2)
    out_v = out_v2.reshape(1, N_HEADS, S_OUT, HEAD_DIM)
    return out_k, out_v


# speedup vs baseline: 20.1735x; 20.1735x over previous
import jax
import jax.numpy as jnp
from jax import lax
from jax.experimental import pallas as pl
from jax.experimental.pallas import tpu as pltpu
from jax.experimental.pallas import tpu_sc as plsc

N_HEADS = 16
HEAD_DIM = 128
N_TRAIN = 2048
S_NEW = 8192
S_OUT = N_TRAIN + S_NEW
HALF = S_OUT // 2


def _tc_assemble_k(tk_ref, nk_ref, ok_ref):
    ok_ref[0, 0, :N_TRAIN, :] = tk_ref[0, 0]
    ok_ref[0, 0, N_TRAIN:, :] = nk_ref[0, 0]


CHUNK = 256
N_CHUNKS_W = HALF // CHUNK


def _staged_copy(srcs, dsts, bufs, sems_in, sems_out):
    n = len(srcs)
    din = [None] * n
    dout = [None] * n
    din[0] = pltpu.async_copy(srcs[0], bufs[0], sems_in[0])
    if n > 1:
        din[1] = pltpu.async_copy(srcs[1], bufs[1], sems_in[1])
    for j in range(n):
        b = j % 2
        din[j].wait()
        dout[j] = pltpu.async_copy(bufs[b], dsts[j], sems_out[b])
        if j + 2 < n:
            dout[j].wait()
            din[j + 2] = pltpu.async_copy(srcs[j + 2], bufs[b], sems_in[b])
    for j in range(max(n - 2, 0), n):
        dout[j].wait()


def _sc_assemble_v(tv_hbm, nv_hbm, out_hbm, buf0, buf1, si0, si1, so0, so1):
    c = lax.axis_index("c")
    s = lax.axis_index("s")
    w = s * 2 + c
    h = w // 2
    even = (w % 2) == 0
    bufs = (buf0, buf1)
    sems_in = (si0, si1)
    sems_out = (so0, so1)

    @pl.when(even)
    def _():
        srcs = []
        dsts = []
        for j in range(N_CHUNKS_W):
            if j < N_TRAIN // CHUNK:
                srcs.append(tv_hbm.at[pl.ds(h * N_TRAIN + j * CHUNK, CHUNK), :])
            else:
                srcs.append(nv_hbm.at[
                    pl.ds(h * S_NEW + (j - N_TRAIN // CHUNK) * CHUNK, CHUNK), :])
            dsts.append(out_hbm.at[pl.ds(h * S_OUT + j * CHUNK, CHUNK), :])
        _staged_copy(srcs, dsts, bufs, sems_in, sems_out)

    @pl.when(jnp.logical_not(even))
    def _():
        srcs = [nv_hbm.at[pl.ds(h * S_NEW + (HALF - N_TRAIN) + j * CHUNK, CHUNK), :]
                for j in range(N_CHUNKS_W)]
        dsts = [out_hbm.at[pl.ds(h * S_OUT + HALF + j * CHUNK, CHUNK), :]
                for j in range(N_CHUNKS_W)]
        _staged_copy(srcs, dsts, bufs, sems_in, sems_out)


def kernel(new_keys, new_values, trainable_keys, trainable_values,
           mem_keys, mem_values, new_seq_ids):
    del mem_keys, mem_values, new_seq_ids

    train_spec = pl.BlockSpec((1, 1, N_TRAIN, HEAD_DIM), lambda h: (0, h, 0, 0))
    new_spec = pl.BlockSpec((1, 1, S_NEW, HEAD_DIM), lambda h: (0, h, 0, 0))
    out_spec = pl.BlockSpec((1, 1, S_OUT, HEAD_DIM), lambda h: (0, h, 0, 0))
    out_shape = jax.ShapeDtypeStruct((1, N_HEADS, S_OUT, HEAD_DIM), jnp.float32)
    out_k = pl.pallas_call(
        _tc_assemble_k,
        grid=(N_HEADS,),
        in_specs=[train_spec, new_spec],
        out_specs=out_spec,
        out_shape=out_shape,
    )(trainable_keys, new_keys)

    tv2 = trainable_values.reshape(N_HEADS * N_TRAIN, HEAD_DIM)
    nv2 = new_values.reshape(N_HEADS * S_NEW, HEAD_DIM)
    out_v2 = pl.kernel(
        _sc_assemble_v,
        out_type=jax.ShapeDtypeStruct((N_HEADS * S_OUT, HEAD_DIM), jnp.float32),
        mesh=plsc.VectorSubcoreMesh(core_axis_name="c", subcore_axis_name="s"),
        scratch_types=[
            pltpu.VMEM((CHUNK, HEAD_DIM), jnp.float32),
            pltpu.VMEM((CHUNK, HEAD_DIM), jnp.float32),
            pltpu.SemaphoreType.DMA,
            pltpu.SemaphoreType.DMA,
            pltpu.SemaphoreType.DMA,
            pltpu.SemaphoreType.DMA,
        ],
    )(tv2, nv2)
    out_v = out_v2.reshape(1, N_HEADS, S_OUT, HEAD_DIM)
    return out_k, out_v
